# baseline (speedup 1.0000x reference)
import functools

import jax
import jax.numpy as jnp
from jax import lax
from jax.experimental import pallas as pl
from jax.experimental.pallas import tpu as pltpu
from jax.experimental.pallas import tpu_sc as plsc

NN = 100000
EE = 6400000
GG = 1024
NP = 100352
NC, NS = 2, 16
NW = NC * NS
RPT = NP // NS

EC = 128
KJ = 8
GRP = 1024
NGRP = EE // GRP
GQ, GR = NGRP // NW, NGRP % NW
KB = 1
NQ = NGRP // KB
PQ, PR = NQ // NW, NQ % NW

NB = NP // EC
BQ, BR = NB // NW, NB % NW

GPW = GG // NW
CH = 32

RB = 2048
NBLK = NP // RB

_mesh = plsc.VectorSubcoreMesh(
    core_axis_name="c", subcore_axis_name="s", num_cores=NC, num_subcores=NS)


def _worker(c, s):
    return c * NS + s


@functools.partial(
    pl.kernel, mesh=_mesh,
    out_type=[jax.ShapeDtypeStruct((NC, NP), jnp.float32),
              jax.ShapeDtypeStruct((NC, 2048), jnp.float32)],
    scratch_types=[
        pltpu.VMEM((GRP,), jnp.int32),
        pltpu.VMEM((1, EC), jnp.int32),
        pltpu.VMEM((GRP,), jnp.float32),
        pltpu.VMEM_SHARED((NP,), jnp.float32),
        pltpu.VMEM_SHARED((2048,), jnp.float32),
    ])
def _sc_histograms(dst1d, batch2d, zeros1, out_deg, out_bc,
                   didx, bidx, ones, dega, bca):
    c = lax.axis_index("c")
    s = lax.axis_index("s")
    w = _worker(c, s)
    pltpu.sync_copy(zeros1.at[pl.ds(0, RPT)], dega.at[pl.ds(s * RPT, RPT)])
    pltpu.sync_copy(zeros1.at[pl.ds(0, 128)], bca.at[pl.ds(s * 128, 128)])

    def ones_body(i, carry):
        ones[pl.ds(i * 16, 16)] = jnp.ones((16,), jnp.float32)
        return carry

    lax.fori_loop(0, GRP // 16, ones_body, 0)
    plsc.subcore_barrier()

    base = w * PQ + jnp.minimum(w, PR)
    n_g = PQ + jnp.where(w < PR, 1, 0)

    def edge_body(g, carry):
        pltpu.sync_copy(dst1d.at[pl.ds((base + g) * GRP, GRP)], didx)
        pltpu.sync_copy(ones, dega.at[didx], add=True)
        return carry

    lax.fori_loop(0, n_g, edge_body, 0)

    bbase = w * BQ + jnp.minimum(w, BR)
    n_b = BQ + jnp.where(w < BR, 1, 0)

    def batch_body(r, carry):
        pltpu.sync_copy(batch2d.at[pl.ds(bbase + r, 1)], bidx)
        pltpu.sync_copy(ones.at[pl.ds(0, EC)], bca.at[bidx.at[0]], add=True)
        return carry

    lax.fori_loop(0, n_b, batch_body, 0)

    plsc.subcore_barrier()
    pltpu.sync_copy(dega.at[pl.ds(s * RPT, RPT)],
                    out_deg.at[c, pl.ds(s * RPT, RPT)])
    pltpu.sync_copy(bca.at[pl.ds(s * 128, 128)],
                    out_bc.at[c, pl.ds(s * 128, 128)])


def _make_agg(F):
    @functools.partial(
        pl.kernel, mesh=_mesh,
        out_type=jax.ShapeDtypeStruct((NC, NP, F), jnp.float32),
        compiler_params=pltpu.CompilerParams(use_tc_tiling_on_sc=False),
        scratch_types=(
            [pltpu.VMEM((GRP,), jnp.int32)] * KB
            + [pltpu.VMEM((GRP,), jnp.int32)] * KB
            + [pltpu.VMEM((GRP, F), jnp.float32)] * KB
            + [pltpu.VMEM_SHARED((NP, F), jnp.float32)]
            + [pltpu.SemaphoreType.DMA] * (KB + 2)
        ))
    def agg(t_hbm, src1d, dst1d, zrows, out, *sc):
        sidx = sc[0:KB]
        didx = sc[KB:2 * KB]
        rows = sc[2 * KB:3 * KB]
        acc = sc[3 * KB]
        gsem = sc[3 * KB + 1:3 * KB + 1 + KB]
        isem = sc[3 * KB + 1 + KB]
        ssem = sc[3 * KB + 2 + KB]
        c = lax.axis_index("c")
        s = lax.axis_index("s")
        w = _worker(c, s)
        pltpu.sync_copy(zrows, acc.at[pl.ds(s * RPT, RPT)])
        plsc.subcore_barrier()

        base = w * PQ + jnp.minimum(w, PR)
        n_p = PQ + jnp.where(w < PR, 1, 0)

        def body(p, carry):
            e0 = (base + p) * KB * GRP
            for k in range(KB):
                pltpu.sync_copy(src1d.at[pl.ds(e0 + k * GRP, GRP)], sidx[k])
                pltpu.sync_copy(dst1d.at[pl.ds(e0 + k * GRP, GRP)], didx[k])
            for k in range(KB):
                pltpu.async_copy(t_hbm.at[sidx[k]], rows[k], gsem[k]).wait()
                pltpu.sync_copy(rows[k], acc.at[didx[k]], add=True)
            return carry

        lax.fori_loop(0, n_p, body, 0)
        plsc.subcore_barrier()
        pltpu.sync_copy(acc.at[pl.ds(s * RPT, RPT)],
                        out.at[c, pl.ds(s * RPT, RPT)])

    return agg


_agg16 = _make_agg(16)


@functools.partial(
    pl.kernel, mesh=_mesh,
    out_type=jax.ShapeDtypeStruct((GG, 48), jnp.float32),
    scratch_types=[
        pltpu.VMEM((GPW,), jnp.int32),
        pltpu.VMEM((GPW,), jnp.int32),
        pltpu.VMEM((CH, 48), jnp.float32),
        pltpu.VMEM((GPW, 48), jnp.float32),
        pltpu.SemaphoreType.DMA,
    ])
def _sc_pool(h3, starts, ends, out, sv, ev, buf, res, sem):
    c = lax.axis_index("c")
    s = lax.axis_index("s")
    w = _worker(c, s)
    pltpu.sync_copy(starts.at[pl.ds(w * GPW, GPW)], sv)
    pltpu.sync_copy(ends.at[pl.ds(w * GPW, GPW)], ev)
    neg = jnp.full((16,), -jnp.inf, jnp.float32)

    for half in range(GPW // 16):
        svec = sv[pl.ds(half * 16, 16)]
        evec = ev[pl.ds(half * 16, 16)]
        for j in range(16):
            st = svec[j]
            en = evec[j]
            al = st - lax.rem(st, 8)
            n_ch = (en - al + (CH - 1)) // CH

            def chunk(k, acc3):
                off = pl.multiple_of(al + k * CH, 8)
                pltpu.sync_copy(h3.at[pl.ds(off, CH)], buf)
                r_lo = jnp.maximum(st - off, 0)
                r_hi = jnp.minimum(en - off, CH)

                def rowmax(r, a):
                    return (jnp.maximum(a[0], buf[r, pl.ds(0, 16)]),
                            jnp.maximum(a[1], buf[r, pl.ds(16, 16)]),
                            jnp.maximum(a[2], buf[r, pl.ds(32, 16)]))

                return lax.fori_loop(r_lo, r_hi, rowmax, acc3)

            m0, m1, m2 = lax.fori_loop(0, n_ch, chunk, (neg, neg, neg))
            res[half * 16 + j, pl.ds(0, 16)] = m0
            res[half * 16 + j, pl.ds(16, 16)] = m1
            res[half * 16 + j, pl.ds(32, 16)] = m2

    pltpu.sync_copy(res, out.at[pl.ds(w * GPW, GPW)])


def _norm_body(degp, x, dinv_o, t1_o):
    degc = jnp.transpose(degp[...])
    deg = jnp.sum(degc, axis=1, keepdims=True) + 1.0
    dinv = lax.rsqrt(deg)
    dinv_o[...] = dinv
    t1_o[...] = jnp.concatenate(
        [dinv * x[...], jnp.zeros((RB, 14), jnp.float32)], axis=1)


def _tc_norm(degp, x):
    return pl.pallas_call(
        _norm_body,
        grid=(NBLK,),
        in_specs=[pl.BlockSpec((NC, RB), lambda i: (0, i)),
                  pl.BlockSpec((RB, 2), lambda i: (i, 0))],
        out_specs=[pl.BlockSpec((RB, 1), lambda i: (i, 0)),
                   pl.BlockSpec((RB, 16), lambda i: (i, 0))],
        out_shape=[jax.ShapeDtypeStruct((NP, 1), jnp.float32),
                   jax.ShapeDtypeStruct((NP, 16), jnp.float32)],
    )(degp, x)


def _starts_body(bcp, starts_o, ends_o):
    counts = bcp[0:1, 0:GG] + bcp[1:2, 0:GG]
    r = lax.broadcasted_iota(jnp.int32, (GG, GG), 0)
    col = lax.broadcasted_iota(jnp.int32, (GG, GG), 1)
    tri = (r < col).astype(jnp.float32)
    st = jnp.dot(counts, tri, preferred_element_type=jnp.float32)
    starts_o[...] = st.astype(jnp.int32)
    ends_o[...] = (st + counts).astype(jnp.int32)


def _tc_starts(bcp):
    return pl.pallas_call(
        _starts_body,
        out_shape=[jax.ShapeDtypeStruct((1, GG), jnp.int32),
                   jax.ShapeDtypeStruct((1, GG), jnp.int32)],
    )(bcp)


def _layer_body(sp, t, dinv, W, b, out):
    u = dinv[...] * (sp[0] + sp[1] + t[...])
    h = jnp.maximum(jnp.dot(u, W[...], preferred_element_type=jnp.float32)
                    + b[...], 0.0)
    out[...] = dinv[...] * h


def _tc_layer(sp, t, dinv, W, b, F, FO):
    return pl.pallas_call(
        _layer_body,
        grid=(NBLK,),
        in_specs=[pl.BlockSpec((NC, RB, F), lambda i: (0, i, 0)),
                  pl.BlockSpec((RB, F), lambda i: (i, 0)),
                  pl.BlockSpec((RB, 1), lambda i: (i, 0)),
                  pl.BlockSpec((F, FO), lambda i: (0, 0)),
                  pl.BlockSpec((1, FO), lambda i: (0, 0))],
        out_specs=pl.BlockSpec((RB, FO), lambda i: (i, 0)),
        out_shape=jax.ShapeDtypeStruct((NP, FO), jnp.float32),
    )(sp, t, dinv, W, b)


def _layer2_body(sp, t, dinv, W, b, out_a, out_b):
    u = dinv[...] * (sp[0] + sp[1] + t[...])
    h = jnp.maximum(jnp.dot(u, W[...], preferred_element_type=jnp.float32)
                    + b[...], 0.0)
    t3 = dinv[...] * h
    out_a[...] = t3[:, 0:16]
    out_b[...] = t3[:, 16:32]


def _tc_layer2(sp, t, dinv, W, b):
    return pl.pallas_call(
        _layer2_body,
        grid=(NBLK,),
        in_specs=[pl.BlockSpec((NC, RB, 16), lambda i: (0, i, 0)),
                  pl.BlockSpec((RB, 16), lambda i: (i, 0)),
                  pl.BlockSpec((RB, 1), lambda i: (i, 0)),
                  pl.BlockSpec((16, 32), lambda i: (0, 0)),
                  pl.BlockSpec((1, 32), lambda i: (0, 0))],
        out_specs=[pl.BlockSpec((RB, 16), lambda i: (i, 0)),
                   pl.BlockSpec((RB, 16), lambda i: (i, 0))],
        out_shape=[jax.ShapeDtypeStruct((NP, 16), jnp.float32),
                   jax.ShapeDtypeStruct((NP, 16), jnp.float32)],
    )(sp, t, dinv, W, b)


def _layer3_body(spa, spb, ta, tb, dinv, W, b, out):
    ua = dinv[...] * (spa[0] + spa[1] + ta[...])
    ub = dinv[...] * (spb[0] + spb[1] + tb[...])
    h = (jnp.dot(ua, W[0:16, :], preferred_element_type=jnp.float32)
         + jnp.dot(ub, W[16:32, :], preferred_element_type=jnp.float32)
         + b[...])
    out[...] = jnp.maximum(h, 0.0)


def _tc_layer3(spa, spb, ta, tb, dinv, W, b):
    return pl.pallas_call(
        _layer3_body,
        grid=(NBLK,),
        in_specs=[pl.BlockSpec((NC, RB, 16), lambda i: (0, i, 0)),
                  pl.BlockSpec((NC, RB, 16), lambda i: (0, i, 0)),
                  pl.BlockSpec((RB, 16), lambda i: (i, 0)),
                  pl.BlockSpec((RB, 16), lambda i: (i, 0)),
                  pl.BlockSpec((RB, 1), lambda i: (i, 0)),
                  pl.BlockSpec((32, 48), lambda i: (0, 0)),
                  pl.BlockSpec((1, 48), lambda i: (0, 0))],
        out_specs=pl.BlockSpec((RB, 48), lambda i: (i, 0)),
        out_shape=jax.ShapeDtypeStruct((NP, 48), jnp.float32),
    )(spa, spb, ta, tb, dinv, W, b)


def _head_body(g, Wl1, bl1, Wl2, bl2, out):
    h = jnp.maximum(jnp.dot(g[...], Wl1[...],
                            preferred_element_type=jnp.float32) + bl1[...], 0.0)
    out[...] = jnp.dot(h, Wl2[...],
                       preferred_element_type=jnp.float32) + bl2[...]


def _tc_head(g, Wl1, bl1, Wl2, bl2):
    return pl.pallas_call(
        _head_body,
        out_shape=jax.ShapeDtypeStruct((GG, 10), jnp.float32),
    )(g, Wl1, bl1, Wl2, bl2)


def kernel(x, edge_index, batch, W1, b1, W2, b2, W3, b3, Wl1, bl1, Wl2, bl2):
    x_p = jnp.pad(x, ((0, NP - NN), (0, 0)))
    src1d = edge_index[0]
    dst1d = edge_index[1]
    batch2d = jnp.pad(batch, (0, NP - NN),
                      constant_values=GG).reshape(NB, EC)
    zeros1 = jnp.zeros((RPT,), jnp.float32)
    zeros16 = jnp.zeros((RPT, 16), jnp.float32)

    degp, bcp = _sc_histograms(dst1d, batch2d, zeros1)
    dinv, t1 = _tc_norm(degp, x_p)
    starts, ends = _tc_starts(bcp)

    sp1 = _agg16(t1, src1d, dst1d, zeros16)
    W1p = jnp.pad(W1, ((0, 14), (0, 0)))
    t2 = _tc_layer(sp1, t1, dinv, W1p, b1.reshape(1, 16), 16, 16)

    sp2 = _agg16(t2, src1d, dst1d, zeros16)
    t3a, t3b = _tc_layer2(sp2, t2, dinv, W2, b2.reshape(1, 32))

    spa = _agg16(t3a, src1d, dst1d, zeros16)
    spb = _agg16(t3b, src1d, dst1d, zeros16)
    h3 = _tc_layer3(spa, spb, t3a, t3b, dinv, W3, b3.reshape(1, 48))

    pooled = _sc_pool(h3, starts.reshape(GG), ends.reshape(GG))
    return _tc_head(pooled, Wl1, bl1.reshape(1, 24), Wl2, bl2.reshape(1, 10))


# speedup vs baseline: 1.2717x; 1.2717x over previous
import functools

import jax
import jax.numpy as jnp
from jax import lax
from jax.experimental import pallas as pl
from jax.experimental.pallas import tpu as pltpu
from jax.experimental.pallas import tpu_sc as plsc

NN = 100000
EE = 6400000
GG = 1024
NP = 100352
NC, NS = 2, 16
NW = NC * NS
RPT = NP // NS

EC = 128
KJ = 8
GRP = 1024
NGRP = EE // GRP
GQ, GR = NGRP // NW, NGRP % NW
KB = 1
NQ = NGRP // KB
PQ, PR = NQ // NW, NQ % NW

NB = NP // EC
BQ, BR = NB // NW, NB % NW

GPW = GG // NW
CH = 32

RB = 2048
NBLK = NP // RB

_mesh = plsc.VectorSubcoreMesh(
    core_axis_name="c", subcore_axis_name="s", num_cores=NC, num_subcores=NS)


def _worker(c, s):
    return c * NS + s


@functools.partial(
    pl.kernel, mesh=_mesh,
    out_type=[jax.ShapeDtypeStruct((NC, NP), jnp.float32),
              jax.ShapeDtypeStruct((NC, 2048), jnp.float32)],
    scratch_types=[
        pltpu.VMEM((GRP,), jnp.int32),
        pltpu.VMEM((1, EC), jnp.int32),
        pltpu.VMEM((GRP,), jnp.float32),
        pltpu.VMEM_SHARED((NP,), jnp.float32),
        pltpu.VMEM_SHARED((2048,), jnp.float32),
    ])
def _sc_histograms(dst1d, batch2d, zeros1, out_deg, out_bc,
                   didx, bidx, ones, dega, bca):
    c = lax.axis_index("c")
    s = lax.axis_index("s")
    w = _worker(c, s)
    pltpu.sync_copy(zeros1.at[pl.ds(0, RPT)], dega.at[pl.ds(s * RPT, RPT)])
    pltpu.sync_copy(zeros1.at[pl.ds(0, 128)], bca.at[pl.ds(s * 128, 128)])

    def ones_body(i, carry):
        ones[pl.ds(i * 16, 16)] = jnp.ones((16,), jnp.float32)
        return carry

    lax.fori_loop(0, GRP // 16, ones_body, 0)
    plsc.subcore_barrier()

    base = w * PQ + jnp.minimum(w, PR)
    n_g = PQ + jnp.where(w < PR, 1, 0)

    def edge_body(g, carry):
        pltpu.sync_copy(dst1d.at[pl.ds((base + g) * GRP, GRP)], didx)
        pltpu.sync_copy(ones, dega.at[didx], add=True)
        return carry

    lax.fori_loop(0, n_g, edge_body, 0)

    bbase = w * BQ + jnp.minimum(w, BR)
    n_b = BQ + jnp.where(w < BR, 1, 0)

    def batch_body(r, carry):
        pltpu.sync_copy(batch2d.at[pl.ds(bbase + r, 1)], bidx)
        pltpu.sync_copy(ones.at[pl.ds(0, EC)], bca.at[bidx.at[0]], add=True)
        return carry

    lax.fori_loop(0, n_b, batch_body, 0)

    plsc.subcore_barrier()
    pltpu.sync_copy(dega.at[pl.ds(s * RPT, RPT)],
                    out_deg.at[c, pl.ds(s * RPT, RPT)])
    pltpu.sync_copy(bca.at[pl.ds(s * 128, 128)],
                    out_bc.at[c, pl.ds(s * 128, 128)])


def _make_agg(F):
    @functools.partial(
        pl.kernel, mesh=_mesh,
        out_type=jax.ShapeDtypeStruct((NC, NP, F), jnp.float32),
        compiler_params=pltpu.CompilerParams(use_tc_tiling_on_sc=False),
        scratch_types=[
            pltpu.VMEM((2, GRP), jnp.int32),
            pltpu.VMEM((2, GRP), jnp.int32),
            pltpu.VMEM((GRP, F), jnp.float32),
            pltpu.VMEM_SHARED((NP, F), jnp.float32),
            pltpu.SemaphoreType.DMA,
            pltpu.SemaphoreType.DMA,
        ])
    def agg(t_hbm, src1d, dst1d, zrows, out,
            sidx, didx, rows, acc, isem, gsem):
        c = lax.axis_index("c")
        s = lax.axis_index("s")
        w = _worker(c, s)
        pltpu.sync_copy(zrows, acc.at[pl.ds(s * RPT, RPT)])

        base = w * PQ + jnp.minimum(w, PR)
        n_p = PQ + jnp.where(w < PR, 1, 0)
        e0 = base * GRP
        pltpu.sync_copy(src1d.at[pl.ds(e0, GRP)], sidx.at[0])
        pltpu.sync_copy(dst1d.at[pl.ds(e0, GRP)], didx.at[0])
        plsc.subcore_barrier()

        def body(p, carry):
            b = lax.rem(p, 2)
            nb = 1 - b
            e1 = (base + jnp.minimum(p + 1, n_p - 1)) * GRP
            i0 = pltpu.async_copy(src1d.at[pl.ds(e1, GRP)], sidx.at[nb], isem)
            i1 = pltpu.async_copy(dst1d.at[pl.ds(e1, GRP)], didx.at[nb], isem)
            pltpu.async_copy(t_hbm.at[sidx.at[b]], rows, gsem).wait()
            pltpu.sync_copy(rows, acc.at[didx.at[b]], add=True)
            i0.wait()
            i1.wait()
            return carry

        lax.fori_loop(0, n_p, body, 0)
        plsc.subcore_barrier()
        pltpu.sync_copy(acc.at[pl.ds(s * RPT, RPT)],
                        out.at[c, pl.ds(s * RPT, RPT)])

    return agg


_agg16 = _make_agg(16)


@functools.partial(
    pl.kernel, mesh=_mesh,
    out_type=jax.ShapeDtypeStruct((GG, 48), jnp.float32),
    scratch_types=[
        pltpu.VMEM((GPW,), jnp.int32),
        pltpu.VMEM((GPW,), jnp.int32),
        pltpu.VMEM((CH, 48), jnp.float32),
        pltpu.VMEM((GPW, 48), jnp.float32),
        pltpu.SemaphoreType.DMA,
    ])
def _sc_pool(h3, starts, ends, out, sv, ev, buf, res, sem):
    c = lax.axis_index("c")
    s = lax.axis_index("s")
    w = _worker(c, s)
    pltpu.sync_copy(starts.at[pl.ds(w * GPW, GPW)], sv)
    pltpu.sync_copy(ends.at[pl.ds(w * GPW, GPW)], ev)
    neg = jnp.full((16,), -jnp.inf, jnp.float32)

    for half in range(GPW // 16):
        svec = sv[pl.ds(half * 16, 16)]
        evec = ev[pl.ds(half * 16, 16)]
        for j in range(16):
            st = svec[j]
            en = evec[j]
            al = st - lax.rem(st, 8)
            n_ch = (en - al + (CH - 1)) // CH

            def chunk(k, acc3):
                off = pl.multiple_of(al + k * CH, 8)
                pltpu.sync_copy(h3.at[pl.ds(off, CH)], buf)
                r_lo = jnp.maximum(st - off, 0)
                r_hi = jnp.minimum(en - off, CH)

                def rowmax(r, a):
                    return (jnp.maximum(a[0], buf[r, pl.ds(0, 16)]),
                            jnp.maximum(a[1], buf[r, pl.ds(16, 16)]),
                            jnp.maximum(a[2], buf[r, pl.ds(32, 16)]))

                return lax.fori_loop(r_lo, r_hi, rowmax, acc3)

            m0, m1, m2 = lax.fori_loop(0, n_ch, chunk, (neg, neg, neg))
            res[half * 16 + j, pl.ds(0, 16)] = m0
            res[half * 16 + j, pl.ds(16, 16)] = m1
            res[half * 16 + j, pl.ds(32, 16)] = m2

    pltpu.sync_copy(res, out.at[pl.ds(w * GPW, GPW)])


def _norm_body(degp, x, dinv_o, t1_o):
    degc = jnp.transpose(degp[...])
    deg = jnp.sum(degc, axis=1, keepdims=True) + 1.0
    dinv = lax.rsqrt(deg)
    dinv_o[...] = dinv
    t1_o[...] = jnp.concatenate(
        [dinv * x[...], jnp.zeros((RB, 14), jnp.float32)], axis=1)


def _tc_norm(degp, x):
    return pl.pallas_call(
        _norm_body,
        grid=(NBLK,),
        in_specs=[pl.BlockSpec((NC, RB), lambda i: (0, i)),
                  pl.BlockSpec((RB, 2), lambda i: (i, 0))],
        out_specs=[pl.BlockSpec((RB, 1), lambda i: (i, 0)),
                   pl.BlockSpec((RB, 16), lambda i: (i, 0))],
        out_shape=[jax.ShapeDtypeStruct((NP, 1), jnp.float32),
                   jax.ShapeDtypeStruct((NP, 16), jnp.float32)],
    )(degp, x)


def _starts_body(bcp, starts_o, ends_o):
    counts = bcp[0:1, 0:GG] + bcp[1:2, 0:GG]
    r = lax.broadcasted_iota(jnp.int32, (GG, GG), 0)
    col = lax.broadcasted_iota(jnp.int32, (GG, GG), 1)
    tri = (r < col).astype(jnp.float32)
    st = jnp.dot(counts, tri, preferred_element_type=jnp.float32)
    starts_o[...] = st.astype(jnp.int32)
    ends_o[...] = (st + counts).astype(jnp.int32)


def _tc_starts(bcp):
    return pl.pallas_call(
        _starts_body,
        out_shape=[jax.ShapeDtypeStruct((1, GG), jnp.int32),
                   jax.ShapeDtypeStruct((1, GG), jnp.int32)],
    )(bcp)


def _layer_body(sp, t, dinv, W, b, out):
    u = dinv[...] * (sp[0] + sp[1] + t[...])
    h = jnp.maximum(jnp.dot(u, W[...], preferred_element_type=jnp.float32)
                    + b[...], 0.0)
    out[...] = dinv[...] * h


def _tc_layer(sp, t, dinv, W, b, F, FO):
    return pl.pallas_call(
        _layer_body,
        grid=(NBLK,),
        in_specs=[pl.BlockSpec((NC, RB, F), lambda i: (0, i, 0)),
                  pl.BlockSpec((RB, F), lambda i: (i, 0)),
                  pl.BlockSpec((RB, 1), lambda i: (i, 0)),
                  pl.BlockSpec((F, FO), lambda i: (0, 0)),
                  pl.BlockSpec((1, FO), lambda i: (0, 0))],
        out_specs=pl.BlockSpec((RB, FO), lambda i: (i, 0)),
        out_shape=jax.ShapeDtypeStruct((NP, FO), jnp.float32),
    )(sp, t, dinv, W, b)


def _layer2_body(sp, t, dinv, W, b, out_a, out_b):
    u = dinv[...] * (sp[0] + sp[1] + t[...])
    h = jnp.maximum(jnp.dot(u, W[...], preferred_element_type=jnp.float32)
                    + b[...], 0.0)
    t3 = dinv[...] * h
    out_a[...] = t3[:, 0:16]
    out_b[...] = t3[:, 16:32]


def _tc_layer2(sp, t, dinv, W, b):
    return pl.pallas_call(
        _layer2_body,
        grid=(NBLK,),
        in_specs=[pl.BlockSpec((NC, RB, 16), lambda i: (0, i, 0)),
                  pl.BlockSpec((RB, 16), lambda i: (i, 0)),
                  pl.BlockSpec((RB, 1), lambda i: (i, 0)),
                  pl.BlockSpec((16, 32), lambda i: (0, 0)),
                  pl.BlockSpec((1, 32), lambda i: (0, 0))],
        out_specs=[pl.BlockSpec((RB, 16), lambda i: (i, 0)),
                   pl.BlockSpec((RB, 16), lambda i: (i, 0))],
        out_shape=[jax.ShapeDtypeStruct((NP, 16), jnp.float32),
                   jax.ShapeDtypeStruct((NP, 16), jnp.float32)],
    )(sp, t, dinv, W, b)


def _layer3_body(spa, spb, ta, tb, dinv, W, b, out):
    ua = dinv[...] * (spa[0] + spa[1] + ta[...])
    ub = dinv[...] * (spb[0] + spb[1] + tb[...])
    h = (jnp.dot(ua, W[0:16, :], preferred_element_type=jnp.float32)
         + jnp.dot(ub, W[16:32, :], preferred_element_type=jnp.float32)
         + b[...])
    out[...] = jnp.maximum(h, 0.0)


def _tc_layer3(spa, spb, ta, tb, dinv, W, b):
    return pl.pallas_call(
        _layer3_body,
        grid=(NBLK,),
        in_specs=[pl.BlockSpec((NC, RB, 16), lambda i: (0, i, 0)),
                  pl.BlockSpec((NC, RB, 16), lambda i: (0, i, 0)),
                  pl.BlockSpec((RB, 16), lambda i: (i, 0)),
                  pl.BlockSpec((RB, 16), lambda i: (i, 0)),
                  pl.BlockSpec((RB, 1), lambda i: (i, 0)),
                  pl.BlockSpec((32, 48), lambda i: (0, 0)),
                  pl.BlockSpec((1, 48), lambda i: (0, 0))],
        out_specs=pl.BlockSpec((RB, 48), lambda i: (i, 0)),
        out_shape=jax.ShapeDtypeStruct((NP, 48), jnp.float32),
    )(spa, spb, ta, tb, dinv, W, b)


def _head_body(g, Wl1, bl1, Wl2, bl2, out):
    h = jnp.maximum(jnp.dot(g[...], Wl1[...],
                            preferred_element_type=jnp.float32) + bl1[...], 0.0)
    out[...] = jnp.dot(h, Wl2[...],
                       preferred_element_type=jnp.float32) + bl2[...]


def _tc_head(g, Wl1, bl1, Wl2, bl2):
    return pl.pallas_call(
        _head_body,
        out_shape=jax.ShapeDtypeStruct((GG, 10), jnp.float32),
    )(g, Wl1, bl1, Wl2, bl2)


def kernel(x, edge_index, batch, W1, b1, W2, b2, W3, b3, Wl1, bl1, Wl2, bl2):
    x_p = jnp.pad(x, ((0, NP - NN), (0, 0)))
    src1d = edge_index[0]
    dst1d = edge_index[1]
    batch2d = jnp.pad(batch, (0, NP - NN),
                      constant_values=GG).reshape(NB, EC)
    zeros1 = jnp.zeros((RPT,), jnp.float32)
    zeros16 = jnp.zeros((RPT, 16), jnp.float32)

    degp, bcp = _sc_histograms(dst1d, batch2d, zeros1)
    dinv, t1 = _tc_norm(degp, x_p)
    starts, ends = _tc_starts(bcp)

    sp1 = _agg16(t1, src1d, dst1d, zeros16)
    W1p = jnp.pad(W1, ((0, 14), (0, 0)))
    t2 = _tc_layer(sp1, t1, dinv, W1p, b1.reshape(1, 16), 16, 16)

    sp2 = _agg16(t2, src1d, dst1d, zeros16)
    t3a, t3b = _tc_layer2(sp2, t2, dinv, W2, b2.reshape(1, 32))

    spa = _agg16(t3a, src1d, dst1d, zeros16)
    spb = _agg16(t3b, src1d, dst1d, zeros16)
    h3 = _tc_layer3(spa, spb, t3a, t3b, dinv, W3, b3.reshape(1, 48))

    pooled = _sc_pool(h3, starts.reshape(GG), ends.reshape(GG))
    return _tc_head(pooled, Wl1, bl1.reshape(1, 24), Wl2, bl2.reshape(1, 10))


# speedup vs baseline: 1.2969x; 1.0198x over previous
import functools

import jax
import jax.numpy as jnp
from jax import lax
from jax.experimental import pallas as pl
from jax.experimental.pallas import tpu as pltpu
from jax.experimental.pallas import tpu_sc as plsc

NN = 100000
EE = 6400000
GG = 1024
NP = 100352
NC, NS = 2, 16
NW = NC * NS
RPT = NP // NS

EC = 128
KJ = 8
GRP = 1024
NGRP = EE // GRP
GQ, GR = NGRP // NW, NGRP % NW
KB = 1
NQ = NGRP // KB
PQ, PR = NQ // NW, NQ % NW

NB = NP // EC
BQ, BR = NB // NW, NB % NW

GPW = GG // NW
CH = 32

RB = 2048
NBLK = NP // RB

_mesh = plsc.VectorSubcoreMesh(
    core_axis_name="c", subcore_axis_name="s", num_cores=NC, num_subcores=NS)


def _worker(c, s):
    return c * NS + s


@functools.partial(
    pl.kernel, mesh=_mesh,
    out_type=[jax.ShapeDtypeStruct((NC, NP), jnp.float32),
              jax.ShapeDtypeStruct((NC, 2048), jnp.float32)],
    compiler_params=pltpu.CompilerParams(use_tc_tiling_on_sc=False),
    scratch_types=[
        pltpu.VMEM((2, GRP), jnp.int32),
        pltpu.VMEM((1, EC), jnp.int32),
        pltpu.VMEM((GRP,), jnp.float32),
        pltpu.VMEM_SHARED((NP,), jnp.float32),
        pltpu.VMEM_SHARED((2048,), jnp.float32),
        pltpu.SemaphoreType.DMA,
    ])
def _sc_histograms(dst1d, batch2d, zeros1, out_deg, out_bc,
                   didx, bidx, ones, dega, bca, isem):
    c = lax.axis_index("c")
    s = lax.axis_index("s")
    w = _worker(c, s)
    pltpu.sync_copy(zeros1.at[pl.ds(0, RPT)], dega.at[pl.ds(s * RPT, RPT)])
    pltpu.sync_copy(zeros1.at[pl.ds(0, 128)], bca.at[pl.ds(s * 128, 128)])

    def ones_body(i, carry):
        ones[pl.ds(i * 16, 16)] = jnp.ones((16,), jnp.float32)
        return carry

    lax.fori_loop(0, GRP // 16, ones_body, 0)

    base = w * PQ + jnp.minimum(w, PR)
    n_g = PQ + jnp.where(w < PR, 1, 0)
    pltpu.sync_copy(dst1d.at[pl.ds(base * GRP, GRP)], didx.at[0])
    plsc.subcore_barrier()

    def edge_body(g, carry):
        b = lax.rem(g, 2)
        e1 = (base + jnp.minimum(g + 1, n_g - 1)) * GRP
        i0 = pltpu.async_copy(dst1d.at[pl.ds(e1, GRP)], didx.at[1 - b], isem)
        pltpu.sync_copy(ones, dega.at[didx.at[b]], add=True)
        i0.wait()
        return carry

    lax.fori_loop(0, n_g, edge_body, 0)

    bbase = w * BQ + jnp.minimum(w, BR)
    n_b = BQ + jnp.where(w < BR, 1, 0)

    def batch_body(r, carry):
        pltpu.sync_copy(batch2d.at[pl.ds(bbase + r, 1)], bidx)
        pltpu.sync_copy(ones.at[pl.ds(0, EC)], bca.at[bidx.at[0]], add=True)
        return carry

    lax.fori_loop(0, n_b, batch_body, 0)

    plsc.subcore_barrier()
    pltpu.sync_copy(dega.at[pl.ds(s * RPT, RPT)],
                    out_deg.at[c, pl.ds(s * RPT, RPT)])
    pltpu.sync_copy(bca.at[pl.ds(s * 128, 128)],
                    out_bc.at[c, pl.ds(s * 128, 128)])


def _make_agg(F):
    @functools.partial(
        pl.kernel, mesh=_mesh,
        out_type=jax.ShapeDtypeStruct((NC, NP, F), jnp.float32),
        compiler_params=pltpu.CompilerParams(use_tc_tiling_on_sc=False),
        scratch_types=[
            pltpu.VMEM((2, GRP), jnp.int32),
            pltpu.VMEM((2, GRP), jnp.int32),
            pltpu.VMEM((GRP, F), jnp.float32),
            pltpu.VMEM_SHARED((NP, F), jnp.float32),
            pltpu.SemaphoreType.DMA,
            pltpu.SemaphoreType.DMA,
        ])
    def agg(t_hbm, src1d, dst1d, zrows, out,
            sidx, didx, rows, acc, isem, gsem):
        c = lax.axis_index("c")
        s = lax.axis_index("s")
        w = _worker(c, s)
        pltpu.sync_copy(zrows, acc.at[pl.ds(s * RPT, RPT)])

        base = w * PQ + jnp.minimum(w, PR)
        n_p = PQ + jnp.where(w < PR, 1, 0)
        e0 = base * GRP
        pltpu.sync_copy(src1d.at[pl.ds(e0, GRP)], sidx.at[0])
        pltpu.sync_copy(dst1d.at[pl.ds(e0, GRP)], didx.at[0])
        plsc.subcore_barrier()

        def body(p, carry):
            b = lax.rem(p, 2)
            nb = 1 - b
            e1 = (base + jnp.minimum(p + 1, n_p - 1)) * GRP
            i0 = pltpu.async_copy(src1d.at[pl.ds(e1, GRP)], sidx.at[nb], isem)
            i1 = pltpu.async_copy(dst1d.at[pl.ds(e1, GRP)], didx.at[nb], isem)
            pltpu.async_copy(t_hbm.at[sidx.at[b]], rows, gsem).wait()
            pltpu.sync_copy(rows, acc.at[didx.at[b]], add=True)
            i0.wait()
            i1.wait()
            return carry

        lax.fori_loop(0, n_p, body, 0)
        plsc.subcore_barrier()
        pltpu.sync_copy(acc.at[pl.ds(s * RPT, RPT)],
                        out.at[c, pl.ds(s * RPT, RPT)])

    return agg


_agg16 = _make_agg(16)


@functools.partial(
    pl.kernel, mesh=_mesh,
    out_type=jax.ShapeDtypeStruct((GG, 48), jnp.float32),
    scratch_types=[
        pltpu.VMEM((GPW,), jnp.int32),
        pltpu.VMEM((GPW,), jnp.int32),
        pltpu.VMEM((CH, 48), jnp.float32),
        pltpu.VMEM((GPW, 48), jnp.float32),
        pltpu.SemaphoreType.DMA,
    ])
def _sc_pool(h3, starts, ends, out, sv, ev, buf, res, sem):
    c = lax.axis_index("c")
    s = lax.axis_index("s")
    w = _worker(c, s)
    pltpu.sync_copy(starts.at[pl.ds(w * GPW, GPW)], sv)
    pltpu.sync_copy(ends.at[pl.ds(w * GPW, GPW)], ev)
    neg = jnp.full((16,), -jnp.inf, jnp.float32)

    for half in range(GPW // 16):
        svec = sv[pl.ds(half * 16, 16)]
        evec = ev[pl.ds(half * 16, 16)]
        for j in range(16):
            st = svec[j]
            en = evec[j]
            al = st - lax.rem(st, 8)
            n_ch = (en - al + (CH - 1)) // CH

            def chunk(k, acc3):
                off = pl.multiple_of(al + k * CH, 8)
                pltpu.sync_copy(h3.at[pl.ds(off, CH)], buf)
                r_lo = jnp.maximum(st - off, 0)
                r_hi = jnp.minimum(en - off, CH)

                def rowmax(r, a):
                    return (jnp.maximum(a[0], buf[r, pl.ds(0, 16)]),
                            jnp.maximum(a[1], buf[r, pl.ds(16, 16)]),
                            jnp.maximum(a[2], buf[r, pl.ds(32, 16)]))

                return lax.fori_loop(r_lo, r_hi, rowmax, acc3)

            m0, m1, m2 = lax.fori_loop(0, n_ch, chunk, (neg, neg, neg))
            res[half * 16 + j, pl.ds(0, 16)] = m0
            res[half * 16 + j, pl.ds(16, 16)] = m1
            res[half * 16 + j, pl.ds(32, 16)] = m2

    pltpu.sync_copy(res, out.at[pl.ds(w * GPW, GPW)])


def _norm_body(degp, x, dinv_o, t1_o):
    degc = jnp.transpose(degp[...])
    deg = jnp.sum(degc, axis=1, keepdims=True) + 1.0
    dinv = lax.rsqrt(deg)
    dinv_o[...] = dinv
    t1_o[...] = jnp.concatenate(
        [dinv * x[...], jnp.zeros((RB, 14), jnp.float32)], axis=1)


def _tc_norm(degp, x):
    return pl.pallas_call(
        _norm_body,
        grid=(NBLK,),
        in_specs=[pl.BlockSpec((NC, RB), lambda i: (0, i)),
                  pl.BlockSpec((RB, 2), lambda i: (i, 0))],
        out_specs=[pl.BlockSpec((RB, 1), lambda i: (i, 0)),
                   pl.BlockSpec((RB, 16), lambda i: (i, 0))],
        out_shape=[jax.ShapeDtypeStruct((NP, 1), jnp.float32),
                   jax.ShapeDtypeStruct((NP, 16), jnp.float32)],
    )(degp, x)


def _starts_body(bcp, starts_o, ends_o):
    counts = bcp[0:1, 0:GG] + bcp[1:2, 0:GG]
    r = lax.broadcasted_iota(jnp.int32, (GG, GG), 0)
    col = lax.broadcasted_iota(jnp.int32, (GG, GG), 1)
    tri = (r < col).astype(jnp.float32)
    st = jnp.dot(counts, tri, preferred_element_type=jnp.float32)
    starts_o[...] = st.astype(jnp.int32)
    ends_o[...] = (st + counts).astype(jnp.int32)


def _tc_starts(bcp):
    return pl.pallas_call(
        _starts_body,
        out_shape=[jax.ShapeDtypeStruct((1, GG), jnp.int32),
                   jax.ShapeDtypeStruct((1, GG), jnp.int32)],
    )(bcp)


def _layer_body(sp, t, dinv, W, b, out):
    u = dinv[...] * (sp[0] + sp[1] + t[...])
    h = jnp.maximum(jnp.dot(u, W[...], preferred_element_type=jnp.float32)
                    + b[...], 0.0)
    out[...] = dinv[...] * h


def _tc_layer(sp, t, dinv, W, b, F, FO):
    return pl.pallas_call(
        _layer_body,
        grid=(NBLK,),
        in_specs=[pl.BlockSpec((NC, RB, F), lambda i: (0, i, 0)),
                  pl.BlockSpec((RB, F), lambda i: (i, 0)),
                  pl.BlockSpec((RB, 1), lambda i: (i, 0)),
                  pl.BlockSpec((F, FO), lambda i: (0, 0)),
                  pl.BlockSpec((1, FO), lambda i: (0, 0))],
        out_specs=pl.BlockSpec((RB, FO), lambda i: (i, 0)),
        out_shape=jax.ShapeDtypeStruct((NP, FO), jnp.float32),
    )(sp, t, dinv, W, b)


def _layer2_body(sp, t, dinv, W, b, out_a, out_b):
    u = dinv[...] * (sp[0] + sp[1] + t[...])
    h = jnp.maximum(jnp.dot(u, W[...], preferred_element_type=jnp.float32)
                    + b[...], 0.0)
    t3 = dinv[...] * h
    out_a[...] = t3[:, 0:16]
    out_b[...] = t3[:, 16:32]


def _tc_layer2(sp, t, dinv, W, b):
    return pl.pallas_call(
        _layer2_body,
        grid=(NBLK,),
        in_specs=[pl.BlockSpec((NC, RB, 16), lambda i: (0, i, 0)),
                  pl.BlockSpec((RB, 16), lambda i: (i, 0)),
                  pl.BlockSpec((RB, 1), lambda i: (i, 0)),
                  pl.BlockSpec((16, 32), lambda i: (0, 0)),
                  pl.BlockSpec((1, 32), lambda i: (0, 0))],
        out_specs=[pl.BlockSpec((RB, 16), lambda i: (i, 0)),
                   pl.BlockSpec((RB, 16), lambda i: (i, 0))],
        out_shape=[jax.ShapeDtypeStruct((NP, 16), jnp.float32),
                   jax.ShapeDtypeStruct((NP, 16), jnp.float32)],
    )(sp, t, dinv, W, b)


def _layer3_body(spa, spb, ta, tb, dinv, W, b, out):
    ua = dinv[...] * (spa[0] + spa[1] + ta[...])
    ub = dinv[...] * (spb[0] + spb[1] + tb[...])
    h = (jnp.dot(ua, W[0:16, :], preferred_element_type=jnp.float32)
         + jnp.dot(ub, W[16:32, :], preferred_element_type=jnp.float32)
         + b[...])
    out[...] = jnp.maximum(h, 0.0)


def _tc_layer3(spa, spb, ta, tb, dinv, W, b):
    return pl.pallas_call(
        _layer3_body,
        grid=(NBLK,),
        in_specs=[pl.BlockSpec((NC, RB, 16), lambda i: (0, i, 0)),
                  pl.BlockSpec((NC, RB, 16), lambda i: (0, i, 0)),
                  pl.BlockSpec((RB, 16), lambda i: (i, 0)),
                  pl.BlockSpec((RB, 16), lambda i: (i, 0)),
                  pl.BlockSpec((RB, 1), lambda i: (i, 0)),
                  pl.BlockSpec((32, 48), lambda i: (0, 0)),
                  pl.BlockSpec((1, 48), lambda i: (0, 0))],
        out_specs=pl.BlockSpec((RB, 48), lambda i: (i, 0)),
        out_shape=jax.ShapeDtypeStruct((NP, 48), jnp.float32),
    )(spa, spb, ta, tb, dinv, W, b)


def _head_body(g, Wl1, bl1, Wl2, bl2, out):
    h = jnp.maximum(jnp.dot(g[...], Wl1[...],
                            preferred_element_type=jnp.float32) + bl1[...], 0.0)
    out[...] = jnp.dot(h, Wl2[...],
                       preferred_element_type=jnp.float32) + bl2[...]


def _tc_head(g, Wl1, bl1, Wl2, bl2):
    return pl.pallas_call(
        _head_body,
        out_shape=jax.ShapeDtypeStruct((GG, 10), jnp.float32),
    )(g, Wl1, bl1, Wl2, bl2)


def kernel(x, edge_index, batch, W1, b1, W2, b2, W3, b3, Wl1, bl1, Wl2, bl2):
    x_p = jnp.pad(x, ((0, NP - NN), (0, 0)))
    src1d = edge_index[0]
    dst1d = edge_index[1]
    batch2d = jnp.pad(batch, (0, NP - NN),
                      constant_values=GG).reshape(NB, EC)
    zeros1 = jnp.zeros((RPT,), jnp.float32)
    zeros16 = jnp.zeros((RPT, 16), jnp.float32)

    degp, bcp = _sc_histograms(dst1d, batch2d, zeros1)
    dinv, t1 = _tc_norm(degp, x_p)
    starts, ends = _tc_starts(bcp)

    sp1 = _agg16(t1, src1d, dst1d, zeros16)
    W1p = jnp.pad(W1, ((0, 14), (0, 0)))
    t2 = _tc_layer(sp1, t1, dinv, W1p, b1.reshape(1, 16), 16, 16)

    sp2 = _agg16(t2, src1d, dst1d, zeros16)
    t3a, t3b = _tc_layer2(sp2, t2, dinv, W2, b2.reshape(1, 32))

    spa = _agg16(t3a, src1d, dst1d, zeros16)
    spb = _agg16(t3b, src1d, dst1d, zeros16)
    h3 = _tc_layer3(spa, spb, t3a, t3b, dinv, W3, b3.reshape(1, 48))

    pooled = _sc_pool(h3, starts.reshape(GG), ends.reshape(GG))
    return _tc_head(pooled, Wl1, bl1.reshape(1, 24), Wl2, bl2.reshape(1, 10))
